# Initial kernel scaffold; baseline (speedup 1.0000x reference)
import jax
import jax.numpy as jnp
from jax.experimental import pallas as pl


def kernel(x):
    raise NotImplementedError("write your pallas kernel here")


# speedup vs baseline: 3.6664x; 3.6664x over previous
import jax
import jax.numpy as jnp
from jax.experimental import pallas as pl
from jax.experimental.pallas import tpu as pltpu

_K = 128
_ROWS = 128
_N = 8192


def _col_iota(shape):
    return jax.lax.broadcasted_iota(jnp.int32, shape, dimension=1)


def _cmp_ex(w, d, asc):
    col = _col_iota(w.shape)
    up = (col & d) == 0
    n = w.shape[1]
    partner = jnp.where(up, pltpu.roll(w, n - d, 1), pltpu.roll(w, d, 1))
    mx = jnp.maximum(w, partner)
    mn = jnp.minimum(w, partner)
    return jnp.where(up != asc, mx, mn)


def _topk_body(x_ref, o_ref):
    w = x_ref[...]
    width = _N

    col = _col_iota(w.shape)
    desc = col < (width // 2)
    m = 2
    while m <= 128:
        d = m // 2
        while d >= 1:
            asc = ((col & m) == 0) != desc
            w = _cmp_ex(w, d, asc)
            d //= 2
        m *= 2

    while width > _K:
        width //= 2
        w = jnp.maximum(w[:, :width], w[:, width:])
        col = _col_iota(w.shape)
        if width > _K:
            asc = col >= (width // 2)
        else:
            asc = jnp.zeros(w.shape, dtype=jnp.bool_)
        d = 64
        while d >= 1:
            w = _cmp_ex(w, d, asc)
            d //= 2

    o_ref[...] = w


def kernel(x):
    return pl.pallas_call(
        _topk_body,
        out_shape=jax.ShapeDtypeStruct((_ROWS, _K), jnp.float32),
        in_specs=[pl.BlockSpec((_ROWS, _N), lambda: (0, 0))],
        out_specs=pl.BlockSpec((_ROWS, _K), lambda: (0, 0)),
    )(x)


# speedup vs baseline: 3.8852x; 1.0597x over previous
import jax
import jax.numpy as jnp
from jax.experimental import pallas as pl
from jax.experimental.pallas import tpu as pltpu

_K = 128
_ROWS = 128
_N = 8192
_NBLK = _N // _K


def _lane_iota(shape):
    return jax.lax.broadcasted_iota(jnp.int32, shape, dimension=len(shape) - 1)


def _blk_iota(shape):
    return jax.lax.broadcasted_iota(jnp.int32, shape, dimension=0)


def _cmp_ex(w, d, asc):
    lane = _lane_iota(w.shape)
    up = (lane & d) == 0
    partner = jnp.where(up, pltpu.roll(w, _K - d, 2), pltpu.roll(w, d, 2))
    mx = jnp.maximum(w, partner)
    mn = jnp.minimum(w, partner)
    return jnp.where(up != asc, mx, mn)


def _topk_body(x_ref, o_ref):
    w = jnp.stack(
        [x_ref[:, b * _K:(b + 1) * _K] for b in range(_NBLK)], axis=0
    )

    lane = _lane_iota(w.shape)
    desc = _blk_iota(w.shape) < (_NBLK // 2)
    m = 2
    while m <= _K:
        d = m // 2
        while d >= 1:
            asc = ((lane & m) == 0) != desc
            w = _cmp_ex(w, d, asc)
            d //= 2
        m *= 2

    nblk = _NBLK
    while nblk > 1:
        nblk //= 2
        w = jnp.maximum(w[:nblk], w[nblk:])
        if nblk > 1:
            asc = _blk_iota(w.shape) >= (nblk // 2)
        else:
            asc = jnp.zeros(w.shape, dtype=jnp.bool_)
        d = 64
        while d >= 1:
            w = _cmp_ex(w, d, asc)
            d //= 2

    o_ref[...] = w[0]


def kernel(x):
    return pl.pallas_call(
        _topk_body,
        out_shape=jax.ShapeDtypeStruct((_ROWS, _K), jnp.float32),
        in_specs=[pl.BlockSpec((_ROWS, _N), lambda: (0, 0))],
        out_specs=pl.BlockSpec((_ROWS, _K), lambda: (0, 0)),
    )(x)
